# baseline (speedup 1.0000x reference)
import jax
import jax.numpy as jnp
from jax import lax
from jax.experimental import pallas as pl
from jax.experimental.pallas import tpu as pltpu
from jax.experimental.pallas import tpu_sc as plsc

N = 10000
E = 320000
D = 128
H = 8
DH = 16
FF = 512
NB = 64
SCALE = DH ** -0.5

NC = 2
NS = 16
NW = NC * NS
EPT = E // NW
BE = 80
NBLK = EPT // BE
ROWS_PER_TILE = N // NS

BN = 2000
GRID = N // BN


def _iota16():
    return jnp.arange(16, dtype=jnp.int32)


def _qkv_body(x_ref, wq_ref, wk_ref, wv_ref, bq_ref, bk_ref, bv_ref,
              deg_ref, bt_ref, q_ref, k_ref, v_ref, b16_ref):
    xb = x_ref[...]
    q_ref[...] = (jnp.dot(xb, wq_ref[...], preferred_element_type=jnp.float32)
                  + bq_ref[...]).astype(jnp.bfloat16)
    k_ref[...] = (jnp.dot(xb, wk_ref[...], preferred_element_type=jnp.float32)
                  + bk_ref[...]).astype(jnp.bfloat16)
    v_ref[...] = jnp.dot(xb, wv_ref[...], preferred_element_type=jnp.float32) + bv_ref[...]
    deg = deg_ref[...]
    onehot = (deg == lax.broadcasted_iota(jnp.int32, (BN, NB), 1)).astype(jnp.float32)
    b16_ref[...] = jnp.dot(onehot, bt_ref[...], preferred_element_type=jnp.float32)


def _qkv_call(x, Wq, Wk, Wv, bq, bk, bv, deg2d, bt16):
    row_spec = pl.BlockSpec((BN, D), lambda i: (i, 0))
    full = lambda shape: pl.BlockSpec(shape, lambda i: (0,) * len(shape))
    return pl.pallas_call(
        _qkv_body,
        grid=(GRID,),
        in_specs=[
            row_spec,
            full((D, D)), full((D, D)), full((D, D)),
            full((1, D)), full((1, D)), full((1, D)),
            pl.BlockSpec((BN, 1), lambda i: (i, 0)),
            full((NB, 16)),
        ],
        out_specs=[row_spec, row_spec, row_spec,
                   pl.BlockSpec((BN, 16), lambda i: (i, 0))],
        out_shape=[
            jax.ShapeDtypeStruct((N, D), jnp.bfloat16),
            jax.ShapeDtypeStruct((N, D), jnp.bfloat16),
            jax.ShapeDtypeStruct((N, D), jnp.float32),
            jax.ShapeDtypeStruct((N, 16), jnp.float32),
        ],
    )(x, Wq, Wk, Wv, bq, bk, bv, deg2d, bt16)


NBUF = 3
MAIN_GROUPS = (NBLK - NBUF) // NBUF
EP = [(b, b % NBUF) for b in range(MAIN_GROUPS * NBUF, NBLK)]
FIRED_MAIN = MAIN_GROUPS * NBUF + NBUF - 1


def _scores_body(q_hbm, k_hbm, b16_hbm, erow3_hbm, ecol3_hbm,
                 scores_hbm, tmax_hbm,
                 idxr2, idxc2, qb, kb, bb, sb, mb,
                 q0, q1, q2, k0, k1, k2, b0, b1, b2, o0, o1, o2):
    cid = lax.axis_index("c")
    sid = lax.axis_index("s")
    wid = sid * NC + cid
    ebase = wid * EPT
    iota = _iota16()
    QS = [q0, q1, q2]
    KS = [k0, k1, k2]
    BS = [b0, b1, b2]
    OS = [o0, o1, o2]

    pltpu.sync_copy(erow3_hbm.at[wid], idxr2)
    pltpu.sync_copy(ecol3_hbm.at[wid], idxc2)

    def fire(blk, s):
        pltpu.async_copy(q_hbm.at[idxr2.at[blk]], qb.at[s], QS[s])
        pltpu.async_copy(k_hbm.at[idxc2.at[blk]], kb.at[s], KS[s])
        pltpu.async_copy(b16_hbm.at[idxc2.at[blk]], bb.at[s], BS[s])

    def out_slice(blk):
        return scores_hbm.at[pl.ds((ebase + blk * BE) * H, BE * H)]

    def process(blk, s, mv):
        @pl.when(blk >= NBUF)
        def _():
            pltpu.make_async_copy(sb.at[s], out_slice(blk - NBUF), OS[s]).wait()
        pltpu.make_async_copy(q_hbm.at[idxr2.at[blk]], qb.at[s], QS[s]).wait()
        pltpu.make_async_copy(k_hbm.at[idxc2.at[blk]], kb.at[s], KS[s]).wait()
        pltpu.make_async_copy(b16_hbm.at[idxc2.at[blk]], bb.at[s], BS[s]).wait()

        def grp_body(g, mv2):
            eidx = g * 16 + iota
            for h in range(H):
                acc = jnp.zeros((16,), jnp.float32)
                for j in range(DH // 2):
                    cvec = h * (DH // 2) + ((iota + j) & (DH // 2 - 1))
                    wq = plsc.load_gather(qb.at[s], [eidx, cvec])
                    wk = plsc.load_gather(kb.at[s], [eidx, cvec])
                    qa, qc = plsc.unpack(plsc.bitcast(wq, jnp.bfloat16),
                                         format=plsc.PackFormat.INTERLEAVED,
                                         preferred_element_type=jnp.float32)
                    ka, kc = plsc.unpack(plsc.bitcast(wk, jnp.bfloat16),
                                         format=plsc.PackFormat.INTERLEAVED,
                                         preferred_element_type=jnp.float32)
                    acc = acc + qa * ka + qc * kc
                plsc.store_scatter(sb.at[s], [eidx * H + h], acc * SCALE)
            return mv2

        mv = lax.fori_loop(0, BE // 16, grp_body, mv)

        hi8 = (iota >= 8).astype(jnp.int32)
        col8 = iota & 7

        def bias_body(i, mv2):
            bv = plsc.load_gather(bb.at[s], [2 * i + hi8, col8])
            v = sb[s, pl.ds(i * 16, 16)] + bv
            sb[s, pl.ds(i * 16, 16)] = v
            return jnp.maximum(mv2, v)

        mv = lax.fori_loop(0, (BE * H) // 16, bias_body, mv)
        pltpu.async_copy(sb.at[s], out_slice(blk), OS[s])
        return mv

    for s in range(NBUF):
        fire(s, s)

    def main_body(g, mv):
        for s in range(NBUF):
            blk = g * NBUF + s
            mv = process(blk, s, mv)
            fire(blk + NBUF, s)
        return mv

    mv = lax.fori_loop(0, MAIN_GROUPS, main_body,
                       jnp.full((16,), -3.0e38, jnp.float32))

    for b_, s_ in EP:
        mv = process(b_, s_, mv)
        nxt = b_ + NBUF
        if FIRED_MAIN < nxt <= NBLK - 1:
            fire(nxt, s_)
    for b_, s_ in EP[-NBUF:]:
        pltpu.make_async_copy(sb.at[s_], out_slice(b_), OS[s_]).wait()

    mb[...] = jnp.full((16,), jnp.max(mv), jnp.float32)
    pltpu.sync_copy(mb, tmax_hbm.at[pl.ds(wid * 16, 16)])


def _scores_call(q, k, b16, erow3, ecol3):
    mesh = plsc.VectorSubcoreMesh(core_axis_name="c", subcore_axis_name="s",
                                  num_cores=NC, num_subcores=NS)
    return pl.kernel(
        _scores_body,
        out_type=[
            jax.ShapeDtypeStruct((E * H,), jnp.float32),
            jax.ShapeDtypeStruct((NW * 16,), jnp.float32),
        ],
        mesh=mesh,
        compiler_params=pltpu.CompilerParams(use_tc_tiling_on_sc=False,
                                             needs_layout_passes=False),
        scratch_types=[
            pltpu.VMEM((NBLK, BE), jnp.int32),
            pltpu.VMEM((NBLK, BE), jnp.int32),
            pltpu.VMEM((NBUF, BE, D // 2), jnp.int32),
            pltpu.VMEM((NBUF, BE, D // 2), jnp.int32),
            pltpu.VMEM((NBUF, BE, 16), jnp.float32),
            pltpu.VMEM((NBUF, BE * H), jnp.float32),
            pltpu.VMEM((16,), jnp.float32),
        ] + [pltpu.SemaphoreType.DMA] * (4 * NBUF),
    )(q, k, b16, erow3, ecol3)


BEA = 40
NBLKA = EPT // BEA
NBUFA = 2
MAIN_A = NBLKA // NBUFA - 1
EPA = [(b, b % NBUFA) for b in range(MAIN_A * NBUFA, NBLKA)]


def _agg_body(v_hbm, scores_hbm, tmax_hbm, erow3_hbm, ecol3_hbm,
              zagg_hbm, zsum_hbm,
              agg_hbm, sum_hbm,
              idxr2, idxc2, vb, wb, sb, eb, mxb, agg_sh, sum_sh,
              ss0, ss1, v0, v1, e0, e1, w0, w1):
    cid = lax.axis_index("c")
    sid = lax.axis_index("s")
    wid = sid * NC + cid
    ebase = wid * EPT
    iota = _iota16()
    SS = [ss0, ss1]
    VS = [v0, v1]
    ES = [e0, e1]
    WS = [w0, w1]

    pltpu.sync_copy(erow3_hbm.at[wid], idxr2)
    pltpu.sync_copy(ecol3_hbm.at[wid], idxc2)

    r0 = sid * ROWS_PER_TILE
    pltpu.sync_copy(zagg_hbm.at[pl.ds(r0, ROWS_PER_TILE)],
                    agg_sh.at[pl.ds(r0, ROWS_PER_TILE)])
    pltpu.sync_copy(zsum_hbm.at[pl.ds(r0, ROWS_PER_TILE)],
                    sum_sh.at[pl.ds(r0, ROWS_PER_TILE)])
    pltpu.sync_copy(tmax_hbm, mxb)
    mv = mxb[pl.ds(0, 16)]
    for i in range(1, NW):
        mv = jnp.maximum(mv, mxb[pl.ds(i * 16, 16)])
    gmax = jnp.max(mv)

    plsc.subcore_barrier()

    def sc_slice(blk):
        return scores_hbm.at[pl.ds((ebase + blk * BEA) * H, BEA * H)]

    def fire(blk, s):
        pltpu.async_copy(sc_slice(blk), sb.at[s], SS[s])
        pltpu.async_copy(v_hbm.at[idxc2.at[blk]], vb.at[s], VS[s])

    def process(blk, s):
        pltpu.make_async_copy(sc_slice(blk), sb.at[s], SS[s]).wait()
        pltpu.make_async_copy(v_hbm.at[idxc2.at[blk]], vb.at[s], VS[s]).wait()

        @pl.when(blk >= NBUFA)
        def _():
            pb = blk - NBUFA
            pltpu.make_async_copy(eb.at[s], sum_sh.at[idxr2.at[pb]], ES[s]).wait()
            pltpu.make_async_copy(wb.at[s], agg_sh.at[idxr2.at[pb]], WS[s]).wait()

        def exp_body(i, _):
            s16 = sb[s, pl.ds(i * 16, 16)]
            e16 = jnp.exp(s16 - gmax)
            sb[s, pl.ds(i * 16, 16)] = e16
            hi = (iota >= 8).astype(jnp.int32)
            rowv = 2 * i + hi
            colv = iota & 7
            plsc.store_scatter(eb.at[s], [rowv, colv], e16)
            return 0
        lax.fori_loop(0, (BEA * H) // 16, exp_body, 0)

        def scale_body(p, _):
            ev = sb[s, pl.ds(p * 16, 16)]
            for h in range(H):
                m0 = jnp.full((16,), ev[h], jnp.float32)
                wb[s, 2 * p, pl.ds(h * DH, DH)] = vb[s, 2 * p, pl.ds(h * DH, DH)] * m0
                m1 = jnp.full((16,), ev[H + h], jnp.float32)
                wb[s, 2 * p + 1, pl.ds(h * DH, DH)] = vb[s, 2 * p + 1, pl.ds(h * DH, DH)] * m1
            return 0
        lax.fori_loop(0, BEA // 2, scale_body, 0)

        pltpu.async_copy(eb.at[s], sum_sh.at[idxr2.at[blk]], ES[s], add=True)
        pltpu.async_copy(wb.at[s], agg_sh.at[idxr2.at[blk]], WS[s], add=True)

    for s in range(NBUFA):
        fire(s, s)

    def main_body(g, carry):
        for s in range(NBUFA):
            blk = g * NBUFA + s
            process(blk, s)
            fire(blk + NBUFA, s)
        return carry

    lax.fori_loop(0, MAIN_A, main_body, 0)

    for b_, s_ in EPA:
        process(b_, s_)
    for b_, s_ in EPA:
        pltpu.make_async_copy(eb.at[s_], sum_sh.at[idxr2.at[b_]], ES[s_]).wait()
        pltpu.make_async_copy(wb.at[s_], agg_sh.at[idxr2.at[b_]], WS[s_]).wait()

    plsc.subcore_barrier()

    pltpu.sync_copy(agg_sh.at[pl.ds(r0, ROWS_PER_TILE)],
                    agg_hbm.at[cid, pl.ds(r0, ROWS_PER_TILE), :])
    pltpu.sync_copy(sum_sh.at[pl.ds(r0, ROWS_PER_TILE)],
                    sum_hbm.at[cid, pl.ds(r0, ROWS_PER_TILE), :])


def _agg_call(v, scores, tmax, erow3a, ecol3a, zagg, zsum):
    mesh = plsc.VectorSubcoreMesh(core_axis_name="c", subcore_axis_name="s",
                                  num_cores=NC, num_subcores=NS)
    return pl.kernel(
        _agg_body,
        out_type=[
            jax.ShapeDtypeStruct((NC, N, D), jnp.float32),
            jax.ShapeDtypeStruct((NC, N, H), jnp.float32),
        ],
        mesh=mesh,
        compiler_params=pltpu.CompilerParams(use_tc_tiling_on_sc=False,
                                             needs_layout_passes=False),
        scratch_types=[
            pltpu.VMEM((NBLKA, BEA), jnp.int32),
            pltpu.VMEM((NBLKA, BEA), jnp.int32),
            pltpu.VMEM((NBUFA, BEA, D), jnp.float32),
            pltpu.VMEM((NBUFA, BEA, D), jnp.float32),
            pltpu.VMEM((NBUFA, BEA * H), jnp.float32),
            pltpu.VMEM((NBUFA, BEA, H), jnp.float32),
            pltpu.VMEM((NW * 16,), jnp.float32),
            pltpu.VMEM_SHARED((N, D), jnp.float32),
            pltpu.VMEM_SHARED((N, H), jnp.float32),
        ] + [pltpu.SemaphoreType.DMA] * (4 * NBUFA),
    )(v, scores, tmax, erow3a, ecol3a, zagg, zsum)


def _ln(t, g, b):
    mu = t.mean(-1, keepdims=True)
    var = ((t - mu) ** 2).mean(-1, keepdims=True)
    return (t - mu) / jnp.sqrt(var + 1e-5) * g + b


def _tail_body(x_ref, agg_ref, sum_ref, sel_ref, wo_ref, bo_ref,
               g1_ref, be1_ref, g2_ref, be2_ref,
               w1_ref, bf1_ref, w2_ref, bf2_ref, out_ref):
    a = agg_ref[0] + agg_ref[1]
    s = sum_ref[0] + sum_ref[1]
    inv = 1.0 / jnp.maximum(s, 1e-38)
    bc = jnp.dot(inv, sel_ref[...], preferred_element_type=jnp.float32)
    aggn = a * bc
    out = jnp.dot(aggn, wo_ref[...], preferred_element_type=jnp.float32) + bo_ref[...]
    h1 = _ln(x_ref[...] + out, g1_ref[...], be1_ref[...])
    t = jnp.dot(h1, w1_ref[...], preferred_element_type=jnp.float32) + bf1_ref[...]
    t = 0.5 * t * (1.0 + lax.erf(t * (2.0 ** -0.5)))
    ff = jnp.dot(t, w2_ref[...], preferred_element_type=jnp.float32) + bf2_ref[...]
    out_ref[...] = _ln(h1 + ff, g2_ref[...], be2_ref[...])


def _tail_call(x, agg2, sum2, sel, Wo, bo, g1, be1, g2, be2, W1, bf1, W2, bf2):
    full = lambda shape: pl.BlockSpec(shape, lambda i: (0,) * len(shape))
    return pl.pallas_call(
        _tail_body,
        grid=(GRID,),
        in_specs=[
            pl.BlockSpec((BN, D), lambda i: (i, 0)),
            pl.BlockSpec((NC, BN, D), lambda i: (0, i, 0)),
            pl.BlockSpec((NC, BN, H), lambda i: (0, i, 0)),
            full((H, D)),
            full((D, D)), full((1, D)),
            full((1, D)), full((1, D)), full((1, D)), full((1, D)),
            full((D, FF)), full((1, FF)),
            full((FF, D)), full((1, D)),
        ],
        out_specs=pl.BlockSpec((BN, D), lambda i: (i, 0)),
        out_shape=jax.ShapeDtypeStruct((N, D), jnp.float32),
    )(x, agg2, sum2, sel, Wo, bo, g1, be1, g2, be2, W1, bf1, W2, bf2)


def kernel(x, edge_index, bias_table, deg_bucket, Wq, bq, Wk, bk, Wv, bv,
           Wo, bo, g1, be1, g2, be2, W1, bf1, W2, bf2):
    erow3 = edge_index[0].reshape(NW, NBLK, BE)
    ecol3 = edge_index[1].reshape(NW, NBLK, BE)
    erow3a = edge_index[0].reshape(NW, NBLKA, BEA)
    ecol3a = edge_index[1].reshape(NW, NBLKA, BEA)
    deg2d = deg_bucket.reshape(N, 1)
    bt16 = jnp.pad(bias_table, ((0, 0), (0, 16 - H)))
    q, k, v, b16 = _qkv_call(x, Wq, Wk, Wv,
                             bq.reshape(1, D), bk.reshape(1, D), bv.reshape(1, D),
                             deg2d, bt16)
    qi = lax.bitcast_convert_type(q.reshape(N, D // 2, 2), jnp.int32)
    ki = lax.bitcast_convert_type(k.reshape(N, D // 2, 2), jnp.int32)

    scores, tmax = _scores_call(qi, ki, b16, erow3, ecol3)

    zagg = jnp.zeros((N, D), jnp.float32)
    zsum = jnp.zeros((N, H), jnp.float32)
    agg2, sum2 = _agg_call(v, scores, tmax, erow3a, ecol3a, zagg, zsum)

    sel = jnp.repeat(jnp.eye(H, dtype=jnp.float32), DH, axis=1)
    return _tail_call(x, agg2, sum2, sel,
                      Wo, bo.reshape(1, D),
                      g1.reshape(1, D), be1.reshape(1, D),
                      g2.reshape(1, D), be2.reshape(1, D),
                      W1, bf1.reshape(1, FF), W2, bf2.reshape(1, D))


# speedup vs baseline: 1.1753x; 1.1753x over previous
import jax
import jax.numpy as jnp
from jax import lax
from jax.experimental import pallas as pl
from jax.experimental.pallas import tpu as pltpu
from jax.experimental.pallas import tpu_sc as plsc

N = 10000
E = 320000
D = 128
H = 8
DH = 16
FF = 512
NB = 64
SCALE = DH ** -0.5

NC = 2
NS = 16
NW = NC * NS
EPT = E // NW
BE = 80
NBLK = EPT // BE
ROWS_PER_TILE = N // NS

BN = 2000
GRID = N // BN


def _iota16():
    return jnp.arange(16, dtype=jnp.int32)


def _qkv_body(x_ref, wq_ref, wk_ref, wv_ref, bq_ref, bk_ref, bv_ref,
              deg_ref, bt_ref, q_ref, k_ref, v_ref, b16_ref):
    xb = x_ref[...]
    q_ref[...] = jnp.dot(xb, wq_ref[...], preferred_element_type=jnp.float32) + bq_ref[...]
    k_ref[...] = jnp.dot(xb, wk_ref[...], preferred_element_type=jnp.float32) + bk_ref[...]
    v_ref[...] = jnp.dot(xb, wv_ref[...], preferred_element_type=jnp.float32) + bv_ref[...]
    deg = deg_ref[...]
    onehot = (deg == lax.broadcasted_iota(jnp.int32, (BN, NB), 1)).astype(jnp.float32)
    b16_ref[...] = jnp.dot(onehot, bt_ref[...], preferred_element_type=jnp.float32)


def _qkv_call(x, Wq, Wk, Wv, bq, bk, bv, deg2d, bt16):
    row_spec = pl.BlockSpec((BN, D), lambda i: (i, 0))
    full = lambda shape: pl.BlockSpec(shape, lambda i: (0,) * len(shape))
    return pl.pallas_call(
        _qkv_body,
        grid=(GRID,),
        in_specs=[
            row_spec,
            full((D, D)), full((D, D)), full((D, D)),
            full((1, D)), full((1, D)), full((1, D)),
            pl.BlockSpec((BN, 1), lambda i: (i, 0)),
            full((NB, 16)),
        ],
        out_specs=[row_spec, row_spec, row_spec,
                   pl.BlockSpec((BN, 16), lambda i: (i, 0))],
        out_shape=[
            jax.ShapeDtypeStruct((N, D), jnp.float32),
            jax.ShapeDtypeStruct((N, D), jnp.float32),
            jax.ShapeDtypeStruct((N, D), jnp.float32),
            jax.ShapeDtypeStruct((N, 16), jnp.float32),
        ],
    )(x, Wq, Wk, Wv, bq, bk, bv, deg2d, bt16)


NBUF = 3
MAIN_GROUPS = (NBLK - NBUF) // NBUF
EP = [(b, b % NBUF) for b in range(MAIN_GROUPS * NBUF, NBLK)]
FIRED_MAIN = MAIN_GROUPS * NBUF + NBUF - 1


def _scores_body(q_hbm, k_hbm, b16_hbm, erow3_hbm, ecol3_hbm,
                 scores_hbm, tmax_hbm,
                 idxr2, idxc2, qb, kb, bb, sb, mb,
                 q0, q1, q2, k0, k1, k2, b0, b1, b2, o0, o1, o2):
    cid = lax.axis_index("c")
    sid = lax.axis_index("s")
    wid = sid * NC + cid
    ebase = wid * EPT
    iota = _iota16()
    QS = [q0, q1, q2]
    KS = [k0, k1, k2]
    BS = [b0, b1, b2]
    OS = [o0, o1, o2]

    pltpu.sync_copy(erow3_hbm.at[wid], idxr2)
    pltpu.sync_copy(ecol3_hbm.at[wid], idxc2)

    def fire(blk, s):
        pltpu.async_copy(q_hbm.at[idxr2.at[blk]], qb.at[s], QS[s])
        pltpu.async_copy(k_hbm.at[idxc2.at[blk]], kb.at[s], KS[s])
        pltpu.async_copy(b16_hbm.at[idxc2.at[blk]], bb.at[s], BS[s])

    def out_slice(blk):
        return scores_hbm.at[pl.ds((ebase + blk * BE) * H, BE * H)]

    def process(blk, s, mv):
        @pl.when(blk >= NBUF)
        def _():
            pltpu.make_async_copy(sb.at[s], out_slice(blk - NBUF), OS[s]).wait()
        pltpu.make_async_copy(q_hbm.at[idxr2.at[blk]], qb.at[s], QS[s]).wait()
        pltpu.make_async_copy(k_hbm.at[idxc2.at[blk]], kb.at[s], KS[s]).wait()
        pltpu.make_async_copy(b16_hbm.at[idxc2.at[blk]], bb.at[s], BS[s]).wait()

        def grp_body(g, mv2):
            eidx = g * 16 + iota
            for h in range(H):
                accs = [jnp.zeros((16,), jnp.float32) for _ in range(4)]
                for d in range(DH):
                    cvec = h * DH + ((iota + d) & (DH - 1))
                    qv = plsc.load_gather(qb.at[s], [eidx, cvec])
                    kv = plsc.load_gather(kb.at[s], [eidx, cvec])
                    accs[d & 3] = accs[d & 3] + qv * kv
                acc = (accs[0] + accs[1]) + (accs[2] + accs[3])
                plsc.store_scatter(sb.at[s], [eidx * H + h], acc * SCALE)
            return mv2

        mv = lax.fori_loop(0, BE // 16, grp_body, mv)

        hi8 = (iota >= 8).astype(jnp.int32)
        col8 = iota & 7

        def bias_body(i, mv2):
            bv = plsc.load_gather(bb.at[s], [2 * i + hi8, col8])
            v = sb[s, pl.ds(i * 16, 16)] + bv
            sb[s, pl.ds(i * 16, 16)] = v
            return jnp.maximum(mv2, v)

        mv = lax.fori_loop(0, (BE * H) // 16, bias_body, mv)
        pltpu.async_copy(sb.at[s], out_slice(blk), OS[s])
        return mv

    for s in range(NBUF):
        fire(s, s)

    def main_body(g, mv):
        for s in range(NBUF):
            blk = g * NBUF + s
            mv = process(blk, s, mv)
            fire(blk + NBUF, s)
        return mv

    mv = lax.fori_loop(0, MAIN_GROUPS, main_body,
                       jnp.full((16,), -3.0e38, jnp.float32))

    for b_, s_ in EP:
        mv = process(b_, s_, mv)
        nxt = b_ + NBUF
        if FIRED_MAIN < nxt <= NBLK - 1:
            fire(nxt, s_)
    for b_, s_ in EP[-NBUF:]:
        pltpu.make_async_copy(sb.at[s_], out_slice(b_), OS[s_]).wait()

    mb[...] = jnp.full((16,), jnp.max(mv), jnp.float32)
    pltpu.sync_copy(mb, tmax_hbm.at[pl.ds(wid * 16, 16)])


def _scores_call(q, k, b16, erow3, ecol3):
    mesh = plsc.VectorSubcoreMesh(core_axis_name="c", subcore_axis_name="s",
                                  num_cores=NC, num_subcores=NS)
    return pl.kernel(
        _scores_body,
        out_type=[
            jax.ShapeDtypeStruct((E * H,), jnp.float32),
            jax.ShapeDtypeStruct((NW * 16,), jnp.float32),
        ],
        mesh=mesh,
        compiler_params=pltpu.CompilerParams(use_tc_tiling_on_sc=False,
                                             needs_layout_passes=False),
        scratch_types=[
            pltpu.VMEM((NBLK, BE), jnp.int32),
            pltpu.VMEM((NBLK, BE), jnp.int32),
            pltpu.VMEM((NBUF, BE, D), jnp.float32),
            pltpu.VMEM((NBUF, BE, D), jnp.float32),
            pltpu.VMEM((NBUF, BE, 16), jnp.float32),
            pltpu.VMEM((NBUF, BE * H), jnp.float32),
            pltpu.VMEM((16,), jnp.float32),
        ] + [pltpu.SemaphoreType.DMA] * (4 * NBUF),
    )(q, k, b16, erow3, ecol3)


BEA = 40
NBLKA = EPT // BEA
NBUFA = 2
MAIN_A = NBLKA // NBUFA - 1
EPA = [(b, b % NBUFA) for b in range(MAIN_A * NBUFA, NBLKA)]


def _agg_body(v_hbm, scores_hbm, tmax_hbm, erow3_hbm, ecol3_hbm,
              zagg_hbm, zsum_hbm,
              agg_hbm, sum_hbm,
              idxr2, idxc2, vb, wb, sb, eb, mxb, agg_sh, sum_sh,
              ss0, ss1, v0, v1, e0, e1, w0, w1):
    cid = lax.axis_index("c")
    sid = lax.axis_index("s")
    wid = sid * NC + cid
    ebase = wid * EPT
    iota = _iota16()
    SS = [ss0, ss1]
    VS = [v0, v1]
    ES = [e0, e1]
    WS = [w0, w1]

    pltpu.sync_copy(erow3_hbm.at[wid], idxr2)
    pltpu.sync_copy(ecol3_hbm.at[wid], idxc2)

    r0 = sid * ROWS_PER_TILE
    pltpu.sync_copy(zagg_hbm.at[pl.ds(r0, ROWS_PER_TILE)],
                    agg_sh.at[pl.ds(r0, ROWS_PER_TILE)])
    pltpu.sync_copy(zsum_hbm.at[pl.ds(r0, ROWS_PER_TILE)],
                    sum_sh.at[pl.ds(r0, ROWS_PER_TILE)])
    pltpu.sync_copy(tmax_hbm, mxb)
    mv = mxb[pl.ds(0, 16)]
    for i in range(1, NW):
        mv = jnp.maximum(mv, mxb[pl.ds(i * 16, 16)])
    gmax = jnp.max(mv)

    plsc.subcore_barrier()

    def sc_slice(blk):
        return scores_hbm.at[pl.ds((ebase + blk * BEA) * H, BEA * H)]

    def fire(blk, s):
        pltpu.async_copy(sc_slice(blk), sb.at[s], SS[s])
        pltpu.async_copy(v_hbm.at[idxc2.at[blk]], vb.at[s], VS[s])

    def process(blk, s):
        pltpu.make_async_copy(sc_slice(blk), sb.at[s], SS[s]).wait()
        pltpu.make_async_copy(v_hbm.at[idxc2.at[blk]], vb.at[s], VS[s]).wait()

        @pl.when(blk >= NBUFA)
        def _():
            pb = blk - NBUFA
            pltpu.make_async_copy(eb.at[s], sum_sh.at[idxr2.at[pb]], ES[s]).wait()
            pltpu.make_async_copy(wb.at[s], agg_sh.at[idxr2.at[pb]], WS[s]).wait()

        def exp_body(i, _):
            s16 = sb[s, pl.ds(i * 16, 16)]
            e16 = jnp.exp(s16 - gmax)
            sb[s, pl.ds(i * 16, 16)] = e16
            hi = (iota >= 8).astype(jnp.int32)
            rowv = 2 * i + hi
            colv = iota & 7
            plsc.store_scatter(eb.at[s], [rowv, colv], e16)
            return 0
        lax.fori_loop(0, (BEA * H) // 16, exp_body, 0)

        def scale_body(p, _):
            ev = sb[s, pl.ds(p * 16, 16)]
            for h in range(H):
                m0 = jnp.full((16,), ev[h], jnp.float32)
                wb[s, 2 * p, pl.ds(h * DH, DH)] = vb[s, 2 * p, pl.ds(h * DH, DH)] * m0
                m1 = jnp.full((16,), ev[H + h], jnp.float32)
                wb[s, 2 * p + 1, pl.ds(h * DH, DH)] = vb[s, 2 * p + 1, pl.ds(h * DH, DH)] * m1
            return 0
        lax.fori_loop(0, BEA // 2, scale_body, 0)

        pltpu.async_copy(eb.at[s], sum_sh.at[idxr2.at[blk]], ES[s], add=True)
        pltpu.async_copy(wb.at[s], agg_sh.at[idxr2.at[blk]], WS[s], add=True)

    for s in range(NBUFA):
        fire(s, s)

    def main_body(g, carry):
        for s in range(NBUFA):
            blk = g * NBUFA + s
            process(blk, s)
            fire(blk + NBUFA, s)
        return carry

    lax.fori_loop(0, MAIN_A, main_body, 0)

    for b_, s_ in EPA:
        process(b_, s_)
    for b_, s_ in EPA:
        pltpu.make_async_copy(eb.at[s_], sum_sh.at[idxr2.at[b_]], ES[s_]).wait()
        pltpu.make_async_copy(wb.at[s_], agg_sh.at[idxr2.at[b_]], WS[s_]).wait()

    plsc.subcore_barrier()

    pltpu.sync_copy(agg_sh.at[pl.ds(r0, ROWS_PER_TILE)],
                    agg_hbm.at[cid, pl.ds(r0, ROWS_PER_TILE), :])
    pltpu.sync_copy(sum_sh.at[pl.ds(r0, ROWS_PER_TILE)],
                    sum_hbm.at[cid, pl.ds(r0, ROWS_PER_TILE), :])


def _agg_call(v, scores, tmax, erow3a, ecol3a, zagg, zsum):
    mesh = plsc.VectorSubcoreMesh(core_axis_name="c", subcore_axis_name="s",
                                  num_cores=NC, num_subcores=NS)
    return pl.kernel(
        _agg_body,
        out_type=[
            jax.ShapeDtypeStruct((NC, N, D), jnp.float32),
            jax.ShapeDtypeStruct((NC, N, H), jnp.float32),
        ],
        mesh=mesh,
        compiler_params=pltpu.CompilerParams(use_tc_tiling_on_sc=False,
                                             needs_layout_passes=False),
        scratch_types=[
            pltpu.VMEM((NBLKA, BEA), jnp.int32),
            pltpu.VMEM((NBLKA, BEA), jnp.int32),
            pltpu.VMEM((NBUFA, BEA, D), jnp.float32),
            pltpu.VMEM((NBUFA, BEA, D), jnp.float32),
            pltpu.VMEM((NBUFA, BEA * H), jnp.float32),
            pltpu.VMEM((NBUFA, BEA, H), jnp.float32),
            pltpu.VMEM((NW * 16,), jnp.float32),
            pltpu.VMEM_SHARED((N, D), jnp.float32),
            pltpu.VMEM_SHARED((N, H), jnp.float32),
        ] + [pltpu.SemaphoreType.DMA] * (4 * NBUFA),
    )(v, scores, tmax, erow3a, ecol3a, zagg, zsum)


def _ln(t, g, b):
    mu = t.mean(-1, keepdims=True)
    var = ((t - mu) ** 2).mean(-1, keepdims=True)
    return (t - mu) / jnp.sqrt(var + 1e-5) * g + b


def _tail_body(x_ref, agg_ref, sum_ref, sel_ref, wo_ref, bo_ref,
               g1_ref, be1_ref, g2_ref, be2_ref,
               w1_ref, bf1_ref, w2_ref, bf2_ref, out_ref):
    a = agg_ref[0] + agg_ref[1]
    s = sum_ref[0] + sum_ref[1]
    inv = 1.0 / jnp.maximum(s, 1e-38)
    bc = jnp.dot(inv, sel_ref[...], preferred_element_type=jnp.float32)
    aggn = a * bc
    out = jnp.dot(aggn, wo_ref[...], preferred_element_type=jnp.float32) + bo_ref[...]
    h1 = _ln(x_ref[...] + out, g1_ref[...], be1_ref[...])
    t = jnp.dot(h1, w1_ref[...], preferred_element_type=jnp.float32) + bf1_ref[...]
    t = 0.5 * t * (1.0 + lax.erf(t * (2.0 ** -0.5)))
    ff = jnp.dot(t, w2_ref[...], preferred_element_type=jnp.float32) + bf2_ref[...]
    out_ref[...] = _ln(h1 + ff, g2_ref[...], be2_ref[...])


def _tail_call(x, agg2, sum2, sel, Wo, bo, g1, be1, g2, be2, W1, bf1, W2, bf2):
    full = lambda shape: pl.BlockSpec(shape, lambda i: (0,) * len(shape))
    return pl.pallas_call(
        _tail_body,
        grid=(GRID,),
        in_specs=[
            pl.BlockSpec((BN, D), lambda i: (i, 0)),
            pl.BlockSpec((NC, BN, D), lambda i: (0, i, 0)),
            pl.BlockSpec((NC, BN, H), lambda i: (0, i, 0)),
            full((H, D)),
            full((D, D)), full((1, D)),
            full((1, D)), full((1, D)), full((1, D)), full((1, D)),
            full((D, FF)), full((1, FF)),
            full((FF, D)), full((1, D)),
        ],
        out_specs=pl.BlockSpec((BN, D), lambda i: (i, 0)),
        out_shape=jax.ShapeDtypeStruct((N, D), jnp.float32),
    )(x, agg2, sum2, sel, Wo, bo, g1, be1, g2, be2, W1, bf1, W2, bf2)


def kernel(x, edge_index, bias_table, deg_bucket, Wq, bq, Wk, bk, Wv, bv,
           Wo, bo, g1, be1, g2, be2, W1, bf1, W2, bf2):
    erow3 = edge_index[0].reshape(NW, NBLK, BE)
    ecol3 = edge_index[1].reshape(NW, NBLK, BE)
    erow3a = edge_index[0].reshape(NW, NBLKA, BEA)
    ecol3a = edge_index[1].reshape(NW, NBLKA, BEA)
    deg2d = deg_bucket.reshape(N, 1)
    bt16 = jnp.pad(bias_table, ((0, 0), (0, 16 - H)))
    q, k, v, b16 = _qkv_call(x, Wq, Wk, Wv,
                             bq.reshape(1, D), bk.reshape(1, D), bv.reshape(1, D),
                             deg2d, bt16)
    scores, tmax = _scores_call(q, k, b16, erow3, ecol3)

    zagg = jnp.zeros((N, D), jnp.float32)
    zsum = jnp.zeros((N, H), jnp.float32)
    agg2, sum2 = _agg_call(v, scores, tmax, erow3a, ecol3a, zagg, zsum)

    sel = jnp.repeat(jnp.eye(H, dtype=jnp.float32), DH, axis=1)
    return _tail_call(x, agg2, sum2, sel,
                      Wo, bo.reshape(1, D),
                      g1.reshape(1, D), be1.reshape(1, D),
                      g2.reshape(1, D), be2.reshape(1, D),
                      W1, bf1.reshape(1, FF), W2, bf2.reshape(1, D))
